# Initial kernel scaffold; baseline (speedup 1.0000x reference)
import jax
import jax.numpy as jnp
from jax.experimental import pallas as pl


def kernel(array, mask):
    raise NotImplementedError("write your pallas kernel here")


# speedup vs baseline: 1.0788x; 1.0788x over previous
import functools

import jax
import jax.numpy as jnp
from jax import lax
from jax.experimental import pallas as pl
from jax.experimental.pallas import tpu as pltpu
from jax.experimental.pallas import tpu_sc as plsc

VOCAB = 1000
BATCH = 1024
SEQ = 50
ROWS = BATCH * SEQ
NC = 2
NS = 16
NW = NC * NS
RPW = ROWS // NW
CH = 32
CHW = CH * VOCAB
CHUNKS = RPW // CH

_mesh = plsc.VectorSubcoreMesh(core_axis_name="c", subcore_axis_name="s")


@functools.partial(
    pl.kernel,
    mesh=_mesh,
    out_type=jax.ShapeDtypeStruct((ROWS * VOCAB,), jnp.float32),
    compiler_params=pltpu.CompilerParams(needs_layout_passes=False),
    scratch_types=[
        pltpu.VMEM((RPW,), jnp.int32),
        pltpu.VMEM((RPW,), jnp.float32),
        pltpu.VMEM((CHW,), jnp.float32),
        pltpu.VMEM((CHW,), jnp.float32),
        pltpu.SemaphoreType.DMA,
        pltpu.SemaphoreType.DMA,
    ],
)
def _onehot_sc(idx_hbm, msk_hbm, out_hbm, idx_v, msk_v, buf0, buf1, sem0, sem1):
    wid = lax.axis_index("s") * NC + lax.axis_index("c")
    row0 = wid * RPW

    pltpu.sync_copy(idx_hbm.at[pl.ds(row0, RPW)], idx_v)
    pltpu.sync_copy(msk_hbm.at[pl.ds(row0, RPW)], msk_v)

    zeros16 = jnp.zeros((16,), jnp.float32)

    def zbody(j, carry):
        for k in range(8):
            off = j * 128 + k * 16
            buf0[pl.ds(off, 16)] = zeros16
            buf1[pl.ds(off, 16)] = zeros16
        return carry

    lax.fori_loop(0, CHW // 128, zbody, 0)

    lane = lax.iota(jnp.int32, 16)
    bufs = (buf0, buf1)
    sems = (sem0, sem1)

    def chunk_positions(chunk):
        ps = []
        for k in range(CH // 16):
            iv = idx_v[pl.ds(chunk * CH + k * 16, 16)]
            ps.append((lane + k * 16) * VOCAB + iv)
        return ps

    def cbody(i, carry):
        for b in range(2):
            chunk = i * 2 + b
            buf = bufs[b]
            sem = sems[b]

            @pl.when(chunk >= 2)
            def _():
                prev = chunk - 2
                pltpu.make_async_copy(
                    buf, out_hbm.at[pl.ds((row0 + prev * CH) * VOCAB, CHW)], sem
                ).wait()
                for p in chunk_positions(prev):
                    plsc.store_scatter(buf, [p], zeros16)

            for k, p in enumerate(chunk_positions(chunk)):
                mv = msk_v[pl.ds(chunk * CH + k * 16, 16)]
                plsc.store_scatter(buf, [p], mv)
            pltpu.make_async_copy(
                buf, out_hbm.at[pl.ds((row0 + chunk * CH) * VOCAB, CHW)], sem
            ).start()
        return carry

    lax.fori_loop(0, CHUNKS // 2, cbody, 0)

    for b in range(2):
        chunk = CHUNKS - 2 + b
        pltpu.make_async_copy(
            bufs[b], out_hbm.at[pl.ds((row0 + chunk * CH) * VOCAB, CHW)], sems[b]
        ).wait()


def kernel(array, mask):
    idx = array.reshape(ROWS).astype(jnp.int32)
    msk = mask.reshape(ROWS).astype(jnp.float32)
    out = _onehot_sc(idx, msk)
    return out.reshape(BATCH, SEQ, VOCAB)


# speedup vs baseline: 1.9390x; 1.7973x over previous
import functools

import jax
import jax.numpy as jnp
from jax import lax
from jax.experimental import pallas as pl
from jax.experimental.pallas import tpu as pltpu
from jax.experimental.pallas import tpu_sc as plsc

VOCAB = 1000
BATCH = 1024
SEQ = 50
NC = 2
NS = 16
NW = NC * NS
BPW = BATCH // NW
RPW = BPW * SEQ
RPAD = RPW + 64

_mesh = plsc.VectorSubcoreMesh(core_axis_name="c", subcore_axis_name="s")


@functools.partial(
    pl.kernel,
    mesh=_mesh,
    out_type=jax.ShapeDtypeStruct((BATCH, SEQ, VOCAB), jnp.float32),
    compiler_params=pltpu.CompilerParams(needs_layout_passes=False),
    scratch_types=[
        pltpu.VMEM((RPAD,), jnp.int32),
        pltpu.VMEM((RPAD,), jnp.float32),
        pltpu.VMEM((SEQ, VOCAB), jnp.float32),
        pltpu.VMEM((SEQ, VOCAB), jnp.float32),
        pltpu.SemaphoreType.DMA,
        pltpu.SemaphoreType.DMA,
    ],
)
def _onehot_sc(idx_hbm, msk_hbm, zeros_hbm, out_hbm, idx_v, msk_v, buf0, buf1, sem0, sem1):
    wid = lax.axis_index("s") * NC + lax.axis_index("c")
    b0 = wid * BPW

    pltpu.sync_copy(idx_hbm.at[pl.ds(b0 * SEQ, RPW)], idx_v.at[pl.ds(0, RPW)])
    pltpu.sync_copy(msk_hbm.at[pl.ds(b0 * SEQ, RPW)], msk_v.at[pl.ds(0, RPW)])

    zeros16 = jnp.zeros((16,), jnp.float32)

    pltpu.sync_copy(zeros_hbm, buf0)
    pltpu.sync_copy(zeros_hbm, buf1)

    lane = lax.iota(jnp.int32, 16)
    bufs = (buf0, buf1)
    sems = (sem0, sem1)

    def scatter_chunk(buf, chunk, values16):
        for k in range(4):
            t = lane + k * 16
            iv = idx_v[pl.ds(chunk * SEQ + k * 16, 16)]
            mv = values16(k)
            if k < 3:
                plsc.store_scatter(buf, [t, iv], mv)
            else:
                plsc.store_scatter(buf, [t, iv], mv, mask=t < SEQ)

    def cbody(i, carry):
        for b in range(2):
            chunk = i * 2 + b
            buf = bufs[b]
            sem = sems[b]

            @pl.when(chunk >= 2)
            def _():
                prev = chunk - 2
                pltpu.make_async_copy(
                    buf, out_hbm.at[b0 + prev], sem
                ).wait()
                scatter_chunk(buf, prev, lambda k: zeros16)

            scatter_chunk(
                buf, chunk,
                lambda k: msk_v[pl.ds(chunk * SEQ + k * 16, 16)],
            )
            pltpu.make_async_copy(buf, out_hbm.at[b0 + chunk], sem).start()
        return carry

    lax.fori_loop(0, BPW // 2, cbody, 0)

    for b in range(2):
        chunk = BPW - 2 + b
        pltpu.make_async_copy(bufs[b], out_hbm.at[b0 + chunk], sems[b]).wait()


def kernel(array, mask):
    idx = array.reshape(BATCH * SEQ).astype(jnp.int32)
    msk = mask.reshape(BATCH * SEQ).astype(jnp.float32)
    zeros = jnp.zeros((SEQ, VOCAB), jnp.float32)
    return _onehot_sc(idx, msk, zeros)
